# baseline (speedup 1.0000x reference)
import functools

import jax
import jax.numpy as jnp
from jax.experimental import pallas as pl
from jax.experimental.pallas import tpu as pltpu


def _fused_kernel(*refs, inv_hw, n_streams, c_per_stream):
    x_refs = refs[:n_streams]
    w1_ref, b1_ref, w2_ref, b2_ref, out_ref = refs[n_streams:]

    h = None
    for j, xr in enumerate(x_refs):
        s = jnp.sum(xr[0, 0], axis=-1, keepdims=True)
        hj = jax.lax.dot_general(
            s, w1_ref[j * c_per_stream:(j + 1) * c_per_stream, :],
            dimension_numbers=(((0,), (0,)), ((), ())),
            preferred_element_type=jnp.float32,
        )
        h = hj if h is None else h + hj

    h = jnp.maximum(h * inv_hw + b1_ref[...], 0.0)
    o = jnp.dot(h, w2_ref[...],
                preferred_element_type=jnp.float32) + b2_ref[...]
    out_ref[0] = 2.0 * jax.nn.sigmoid(o) - 1.0


def kernel(x, w1_t, b1_2d, w2_t, b2_2d):
    n, c = x.shape[0], x.shape[1]
    hw = 1
    for d in x.shape[2:]:
        hw *= d
    hidden = w1_t.shape[1]
    out_dim = w2_t.shape[1]

    n_streams = 1
    for cand in (4, 2):
        if c % cand == 0 and (c // cand) % 8 == 0:
            n_streams = cand
            break
    c_per_stream = c // n_streams
    x_g = x.reshape(n, n_streams, c_per_stream, hw)

    body = functools.partial(_fused_kernel, inv_hw=1.0 / float(hw),
                             n_streams=n_streams, c_per_stream=c_per_stream)

    def x_spec(j):
        return pl.BlockSpec((1, 1, c_per_stream, hw),
                            lambda i, j=j: (i, j, 0, 0))

    itemsize = x.dtype.itemsize
    cost = pl.CostEstimate(
        flops=int(n) * int(c) * (int(hw) + 2 * int(hidden))
              + 2 * int(n) * int(hidden) * int(out_dim),
        transcendentals=int(n) * int(out_dim),
        bytes_accessed=int(n) * int(c) * int(hw) * int(itemsize)
                       + int(n) * int(out_dim) * 4,
    )

    out3 = pl.pallas_call(
        body,
        out_shape=jax.ShapeDtypeStruct((n, 1, out_dim), jnp.float32),
        grid=(n,),
        in_specs=[x_spec(j) for j in range(n_streams)] + [
            pl.BlockSpec((c, hidden), lambda i: (0, 0)),
            pl.BlockSpec((1, hidden), lambda i: (0, 0)),
            pl.BlockSpec((hidden, out_dim), lambda i: (0, 0)),
            pl.BlockSpec((1, out_dim), lambda i: (0, 0)),
        ],
        out_specs=pl.BlockSpec((1, 1, out_dim), lambda i: (i, 0, 0)),
        compiler_params=pltpu.CompilerParams(
            dimension_semantics=("parallel",),
            vmem_limit_bytes=48 * 1024 * 1024,
        ),
        cost_estimate=cost,
    )(*([x_g] * n_streams), w1_t, b1_2d, w2_t, b2_2d)

    return out3.reshape(n, out_dim)


# speedup vs baseline: 1.6365x; 1.6365x over previous
import functools

import jax
import jax.numpy as jnp
from jax.experimental import pallas as pl
from jax.experimental.pallas import tpu as pltpu


def _fused_kernel(x_ref, w1_ref, b1_ref, w2_ref, b2_ref, out_ref, *, inv_hw):
    v = x_ref[0]
    while v.ndim > 2:
        v = jnp.sum(v, axis=-2)
    s = jnp.sum(v, axis=-1, keepdims=True)

    h = jax.lax.dot_general(
        s, w1_ref[...],
        dimension_numbers=(((0,), (0,)), ((), ())),
        preferred_element_type=jnp.float32,
    )
    h = jnp.maximum(h * inv_hw + b1_ref[...], 0.0)

    o = jnp.dot(h, w2_ref[...],
                preferred_element_type=jnp.float32) + b2_ref[...]
    out_ref[0] = 2.0 * jax.nn.sigmoid(o) - 1.0


def kernel(x, w1_t, b1_2d, w2_t, b2_2d):
    n, c = x.shape[0], x.shape[1]
    hw = 1
    for d in x.shape[2:]:
        hw *= d
    hidden = w1_t.shape[1]
    out_dim = w2_t.shape[1]

    body = functools.partial(_fused_kernel, inv_hw=1.0 / float(hw))

    nd_tail = len(x.shape) - 1
    x_block = (1,) + tuple(x.shape[1:])

    itemsize = x.dtype.itemsize
    cost = pl.CostEstimate(
        flops=int(n) * int(c) * (int(hw) + 2 * int(hidden))
              + 2 * int(n) * int(hidden) * int(out_dim),
        transcendentals=int(n) * int(out_dim),
        bytes_accessed=int(n) * int(c) * int(hw) * int(itemsize)
                       + int(n) * int(out_dim) * 4,
    )

    out3 = pl.pallas_call(
        body,
        out_shape=jax.ShapeDtypeStruct((n, 1, out_dim), jnp.float32),
        grid=(n,),
        in_specs=[
            pl.BlockSpec(x_block, lambda i: (i,) + (0,) * nd_tail),
            pl.BlockSpec((c, hidden), lambda i: (0, 0)),
            pl.BlockSpec((1, hidden), lambda i: (0, 0)),
            pl.BlockSpec((hidden, out_dim), lambda i: (0, 0)),
            pl.BlockSpec((1, out_dim), lambda i: (0, 0)),
        ],
        out_specs=pl.BlockSpec((1, 1, out_dim), lambda i: (i, 0, 0)),
        compiler_params=pltpu.CompilerParams(
            dimension_semantics=("parallel",),
            vmem_limit_bytes=48 * 1024 * 1024,
        ),
        cost_estimate=cost,
    )(x, w1_t, b1_2d, w2_t, b2_2d)

    return out3.reshape(n, out_dim)


# speedup vs baseline: 7.1044x; 4.3413x over previous
import functools

import jax
import jax.numpy as jnp
from jax.experimental import pallas as pl
from jax.experimental.pallas import tpu as pltpu


def _fused_kernel(x_ref, w1_ref, b1_ref, w2_ref, b2_ref, out_ref, *, inv_hw):
    v = x_ref[0]
    while v.ndim > 2:
        v = jnp.sum(v, axis=0)
    s = jnp.sum(v, axis=0, keepdims=True)

    h = jnp.dot(s, w1_ref[...],
                preferred_element_type=jnp.float32)
    h = jnp.maximum(h * inv_hw + b1_ref[...], 0.0)

    o = jnp.dot(h, w2_ref[...],
                preferred_element_type=jnp.float32) + b2_ref[...]
    out_ref[0] = 2.0 * jax.nn.sigmoid(o) - 1.0


def kernel(x, w1_t, b1_2d, w2_t, b2_2d):
    n, c = x.shape[0], x.shape[1]
    hw = 1
    for d in x.shape[2:]:
        hw *= d
    hidden = w1_t.shape[1]
    out_dim = w2_t.shape[1]

    perm = (0,) + tuple(range(2, x.ndim)) + (1,)
    xt = jax.lax.transpose(x, perm)

    body = functools.partial(_fused_kernel, inv_hw=1.0 / float(hw))

    nd_tail = xt.ndim - 1
    x_block = (1,) + tuple(xt.shape[1:])

    itemsize = x.dtype.itemsize
    cost = pl.CostEstimate(
        flops=int(n) * int(c) * (int(hw) + 2 * int(hidden))
              + 2 * int(n) * int(hidden) * int(out_dim),
        transcendentals=int(n) * int(out_dim),
        bytes_accessed=int(n) * int(c) * int(hw) * int(itemsize)
                       + int(n) * int(out_dim) * 4,
    )

    out3 = pl.pallas_call(
        body,
        out_shape=jax.ShapeDtypeStruct((n, 1, out_dim), jnp.float32),
        grid=(n,),
        in_specs=[
            pl.BlockSpec(x_block, lambda i: (i,) + (0,) * nd_tail),
            pl.BlockSpec((c, hidden), lambda i: (0, 0)),
            pl.BlockSpec((1, hidden), lambda i: (0, 0)),
            pl.BlockSpec((hidden, out_dim), lambda i: (0, 0)),
            pl.BlockSpec((1, out_dim), lambda i: (0, 0)),
        ],
        out_specs=pl.BlockSpec((1, 1, out_dim), lambda i: (i, 0, 0)),
        compiler_params=pltpu.CompilerParams(
            dimension_semantics=("parallel",),
            vmem_limit_bytes=48 * 1024 * 1024,
        ),
        cost_estimate=cost,
    )(xt, w1_t, b1_2d, w2_t, b2_2d)

    return out3.reshape(n, out_dim)
